# Initial kernel scaffold; baseline (speedup 1.0000x reference)
import jax
import jax.numpy as jnp
from jax.experimental import pallas as pl


def kernel(lane_features, suc_0, n_hop_pre_0, pre_0, n_hop_suc_0, suc_1, n_hop_pre_1, pre_1, n_hop_suc_1, W_center, W_pre, W_suc, gn1_gamma, gn1_beta, W_center2, gn2_gamma, gn2_beta):
    raise NotImplementedError("write your pallas kernel here")


# speedup vs baseline: 4.7359x; 4.7359x over previous
import functools

import jax
import jax.numpy as jnp
from jax import lax
from jax.experimental import pallas as pl
from jax.experimental.pallas import tpu as pltpu
from jax.experimental.pallas import tpu_sc as plsc

NUM_SUBCORES = 16
NUM_CORES = 2
CHUNK = 125
NUM_LISTS = 4
NQ = 4
QW = 64


@functools.lru_cache(maxsize=None)
def _build_sc(N, D, E):
    EPT = E // NUM_SUBCORES
    NCH = EPT // CHUNK
    NPAIR = NCH // 2
    TAIL = NCH % 2
    NPAD = -(-N // (NUM_SUBCORES * 128)) * (NUM_SUBCORES * 128)
    RPT = NPAD // NUM_SUBCORES
    ZR = 128
    assert EPT % CHUNK == 0 and RPT % ZR == 0

    mesh = plsc.VectorSubcoreMesh(core_axis_name="c", subcore_axis_name="s")

    @functools.partial(
        pl.kernel,
        mesh=mesh,
        compiler_params=pltpu.CompilerParams(use_tc_tiling_on_sc=False),
        out_type=jax.ShapeDtypeStruct((NUM_LISTS * NQ, NPAD, QW), jnp.float32),
        scratch_types=[
            pltpu.VMEM((NCH, CHUNK), jnp.int32),
            pltpu.VMEM((NCH, CHUNK), jnp.int32),
            pltpu.VMEM((CHUNK, QW), jnp.float32),
            pltpu.VMEM((CHUNK, QW), jnp.float32),
            pltpu.VMEM((ZR, QW), jnp.float32),
            pltpu.VMEM_SHARED((NPAD, QW), jnp.float32),
            pltpu.SemaphoreType.DMA,
            pltpu.SemaphoreType.DMA,
        ],
    )
    def sc_fn(xq, srcs, dsts, zeros_h, out,
              src_v, dst_v, rows_a, rows_b, zero_v, acc_sh, sem_a, sem_b):
        s = lax.axis_index("s")
        c = lax.axis_index("c")
        pltpu.sync_copy(zeros_h, zero_v)

        def run(cc):
            for j in range(NUM_LISTS):
                pltpu.sync_copy(srcs.at[j, s], src_v)
                pltpu.sync_copy(dsts.at[j, s], dst_v)
                for p in range(NQ // NUM_CORES):
                    q = (NQ // NUM_CORES) * cc + p
                    xref = xq.at[q]
                    for z in range(RPT // ZR):
                        pltpu.sync_copy(zero_v, acc_sh.at[pl.ds(s * RPT + z * ZR, ZR)])
                    plsc.subcore_barrier()
                    pltpu.async_copy(xref.at[src_v.at[0]], rows_a, sem_a)

                    def body(i, carry):
                        k0 = 2 * i
                        pltpu.async_copy(xref.at[src_v.at[k0 + 1]], rows_b, sem_b)
                        pltpu.make_async_copy(xref.at[src_v.at[k0]], rows_a, sem_a).wait()
                        pltpu.sync_copy(rows_a, acc_sh.at[dst_v.at[k0]], add=True)
                        if TAIL:
                            pltpu.async_copy(xref.at[src_v.at[k0 + 2]], rows_a, sem_a)
                        else:
                            @pl.when(i + 1 < NPAIR)
                            def _():
                                pltpu.async_copy(xref.at[src_v.at[k0 + 2]], rows_a, sem_a)
                        pltpu.make_async_copy(xref.at[src_v.at[k0 + 1]], rows_b, sem_b).wait()
                        pltpu.sync_copy(rows_b, acc_sh.at[dst_v.at[k0 + 1]], add=True)
                        return carry

                    lax.fori_loop(0, NPAIR, body, 0)
                    if TAIL:
                        pltpu.make_async_copy(xref.at[src_v.at[NCH - 1]], rows_a, sem_a).wait()
                        pltpu.sync_copy(rows_a, acc_sh.at[dst_v.at[NCH - 1]], add=True)
                    plsc.subcore_barrier()
                    pltpu.sync_copy(
                        acc_sh.at[pl.ds(s * RPT, RPT)],
                        out.at[j * NQ + q, pl.ds(s * RPT, RPT)],
                    )
                    plsc.subcore_barrier()

        @pl.when(c == 0)
        def _():
            run(0)

        @pl.when(c == 1)
        def _():
            run(1)

    return sc_fn


def _tc_body(x_ref, a_ref, wcat_ref, w2_ref, g1_ref, b1_ref, g2_ref, b2_ref, o_ref):
    x = x_ref[...]
    xcat = jnp.concatenate([x] + [a_ref[k] for k in range(NUM_LISTS * NQ)], axis=1)
    t = jnp.dot(xcat, wcat_ref[...], preferred_element_type=jnp.float32)
    m = jnp.mean(t, axis=1, keepdims=True)
    v = jnp.mean(jnp.square(t - m), axis=1, keepdims=True)
    h = (t - m) * lax.rsqrt(v + 1e-5) * g1_ref[...] + b1_ref[...]
    h = jnp.maximum(h, 0.0)
    y = jnp.dot(h, w2_ref[...], preferred_element_type=jnp.float32)
    m2 = jnp.mean(y, axis=1, keepdims=True)
    v2 = jnp.mean(jnp.square(y - m2), axis=1, keepdims=True)
    y = (y - m2) * lax.rsqrt(v2 + 1e-5) * g2_ref[...] + b2_ref[...]
    o_ref[...] = jnp.maximum(y + x, 0.0)


@functools.lru_cache(maxsize=None)
def _build_tc(N, D, RB=1000):
    KC = D + NUM_LISTS * NQ * QW
    return pl.pallas_call(
        _tc_body,
        grid=(N // RB,),
        in_specs=[
            pl.BlockSpec((RB, D), lambda i: (i, 0)),
            pl.BlockSpec((NUM_LISTS * NQ, RB, QW), lambda i: (0, i, 0)),
            pl.BlockSpec((KC, D), lambda i: (0, 0)),
            pl.BlockSpec((D, D), lambda i: (0, 0)),
            pl.BlockSpec((1, D), lambda i: (0, 0)),
            pl.BlockSpec((1, D), lambda i: (0, 0)),
            pl.BlockSpec((1, D), lambda i: (0, 0)),
            pl.BlockSpec((1, D), lambda i: (0, 0)),
        ],
        out_specs=pl.BlockSpec((RB, D), lambda i: (i, 0)),
        out_shape=jax.ShapeDtypeStruct((N, D), jnp.float32),
    )


def kernel(lane_features, suc_0, n_hop_pre_0, pre_0, n_hop_suc_0,
           suc_1, n_hop_pre_1, pre_1, n_hop_suc_1,
           W_center, W_pre, W_suc, gn1_gamma, gn1_beta,
           W_center2, gn2_gamma, gn2_beta):
    N, D = lane_features.shape
    E = suc_0.shape[0]
    NB = W_center.shape[0]

    srcs = jnp.stack([n_hop_pre_0, n_hop_suc_0, n_hop_pre_1, n_hop_suc_1])
    dsts = jnp.stack([suc_0, pre_0, suc_1, pre_1])
    srcs = srcs.astype(jnp.int32).reshape(NUM_LISTS, NUM_SUBCORES, -1, CHUNK)
    dsts = dsts.astype(jnp.int32).reshape(NUM_LISTS, NUM_SUBCORES, -1, CHUNK)
    zeros_h = jnp.zeros((128, QW), jnp.float32)

    sc = _build_sc(N, D, E)
    tc = _build_tc(N, D)

    x = lane_features
    for b in range(NB):
        wcat = jnp.concatenate([W_center[b].T, W_pre[0, b].T, W_suc[0, b].T,
                                W_pre[1, b].T, W_suc[1, b].T], axis=0)
        xq = jnp.swapaxes(x.reshape(N, NQ, QW), 0, 1)
        accs = sc(xq, srcs, dsts, zeros_h)
        x = tc(x, accs, wcat, W_center2[b].T,
               gn1_gamma[b][None], gn1_beta[b][None],
               gn2_gamma[b][None], gn2_beta[b][None])
    return x


# speedup vs baseline: 5.6619x; 1.1955x over previous
import functools

import jax
import jax.numpy as jnp
from jax import lax
from jax.experimental import pallas as pl
from jax.experimental.pallas import tpu as pltpu
from jax.experimental.pallas import tpu_sc as plsc

NUM_SUBCORES = 16
NUM_CORES = 2
CHUNK = 125
NUM_LISTS = 4
NQ = 4
QW = 64


@functools.lru_cache(maxsize=None)
def _build_sc(N, D, E):
    EPT = E // NUM_SUBCORES
    NCH = EPT // CHUNK
    NBUF = 5
    NPAD = -(-N // (NUM_SUBCORES * 128)) * (NUM_SUBCORES * 128)
    RPT = NPAD // NUM_SUBCORES
    ZR = 128
    assert EPT % CHUNK == 0 and RPT % ZR == 0 and NCH % NBUF == 0 and NCH >= 4

    mesh = plsc.VectorSubcoreMesh(core_axis_name="c", subcore_axis_name="s")

    @functools.partial(
        pl.kernel,
        mesh=mesh,
        compiler_params=pltpu.CompilerParams(use_tc_tiling_on_sc=False),
        out_type=jax.ShapeDtypeStruct((NUM_LISTS * NQ, NPAD, QW), jnp.float32),
        scratch_types=[
            pltpu.VMEM((NCH, CHUNK), jnp.int32),
            pltpu.VMEM((NCH, CHUNK), jnp.int32),
        ] + [pltpu.VMEM((CHUNK, QW), jnp.float32) for _ in range(NBUF)] + [
            pltpu.VMEM((ZR, QW), jnp.float32),
            pltpu.VMEM_SHARED((NPAD, QW), jnp.float32),
        ] + [pltpu.SemaphoreType.DMA for _ in range(2 * NBUF)],
    )
    def sc_fn(xq, srcs, dsts, zeros_h, out,
              src_v, dst_v, *rest):
        bufs = rest[:NBUF]
        zero_v = rest[NBUF]
        acc_sh = rest[NBUF + 1]
        gsem = rest[NBUF + 2:2 * NBUF + 2]
        ssem = rest[2 * NBUF + 2:]
        s = lax.axis_index("s")
        c = lax.axis_index("c")
        pltpu.sync_copy(zeros_h, zero_v)

        def run(cc):
            for j in range(NUM_LISTS):
                pltpu.sync_copy(srcs.at[j, s], src_v)
                pltpu.sync_copy(dsts.at[j, s], dst_v)
                for p in range(NQ // NUM_CORES):
                    q = (NQ // NUM_CORES) * cc + p
                    xref = xq.at[q]
                    for z in range(RPT // ZR):
                        pltpu.sync_copy(zero_v, acc_sh.at[pl.ds(s * RPT + z * ZR, ZR)])
                    plsc.subcore_barrier()
                    for u in range(3):
                        pltpu.async_copy(xref.at[src_v.at[u]], bufs[u], gsem[u])

                    def body(i, carry):
                        for u in range(NBUF):
                            k = NBUF * i + u
                            pltpu.make_async_copy(
                                xref.at[src_v.at[k]], bufs[u], gsem[u]).wait()
                            pltpu.async_copy(
                                bufs[u], acc_sh.at[dst_v.at[k]], ssem[u], add=True)
                            up = (u + 3) % NBUF

                            @pl.when(k >= 2)
                            def _():
                                pltpu.make_async_copy(
                                    bufs[up], acc_sh.at[dst_v.at[k - 2]],
                                    ssem[up]).wait()

                            @pl.when(k + 3 < NCH)
                            def _():
                                pltpu.async_copy(
                                    xref.at[src_v.at[k + 3]], bufs[up], gsem[up])
                        return carry

                    lax.fori_loop(0, NCH // NBUF, body, 0)
                    for k in (NCH - 2, NCH - 1):
                        u = k % NBUF
                        pltpu.make_async_copy(
                            bufs[u], acc_sh.at[dst_v.at[k]], ssem[u]).wait()
                    plsc.subcore_barrier()
                    pltpu.sync_copy(
                        acc_sh.at[pl.ds(s * RPT, RPT)],
                        out.at[j * NQ + q, pl.ds(s * RPT, RPT)],
                    )
                    plsc.subcore_barrier()

        @pl.when(c == 0)
        def _():
            run(0)

        @pl.when(c == 1)
        def _():
            run(1)

    return sc_fn


def _tc_body(x_ref, a_ref, wcat_ref, w2_ref, g1_ref, b1_ref, g2_ref, b2_ref, o_ref):
    x = x_ref[...]
    xcat = jnp.concatenate([x] + [a_ref[k] for k in range(NUM_LISTS * NQ)], axis=1)
    t = jnp.dot(xcat, wcat_ref[...], preferred_element_type=jnp.float32)
    m = jnp.mean(t, axis=1, keepdims=True)
    v = jnp.mean(jnp.square(t - m), axis=1, keepdims=True)
    h = (t - m) * lax.rsqrt(v + 1e-5) * g1_ref[...] + b1_ref[...]
    h = jnp.maximum(h, 0.0)
    y = jnp.dot(h, w2_ref[...], preferred_element_type=jnp.float32)
    m2 = jnp.mean(y, axis=1, keepdims=True)
    v2 = jnp.mean(jnp.square(y - m2), axis=1, keepdims=True)
    y = (y - m2) * lax.rsqrt(v2 + 1e-5) * g2_ref[...] + b2_ref[...]
    o_ref[...] = jnp.maximum(y + x, 0.0)


@functools.lru_cache(maxsize=None)
def _build_tc(N, D, RB=1000):
    KC = D + NUM_LISTS * NQ * QW
    return pl.pallas_call(
        _tc_body,
        grid=(N // RB,),
        in_specs=[
            pl.BlockSpec((RB, D), lambda i: (i, 0)),
            pl.BlockSpec((NUM_LISTS * NQ, RB, QW), lambda i: (0, i, 0)),
            pl.BlockSpec((KC, D), lambda i: (0, 0)),
            pl.BlockSpec((D, D), lambda i: (0, 0)),
            pl.BlockSpec((1, D), lambda i: (0, 0)),
            pl.BlockSpec((1, D), lambda i: (0, 0)),
            pl.BlockSpec((1, D), lambda i: (0, 0)),
            pl.BlockSpec((1, D), lambda i: (0, 0)),
        ],
        out_specs=pl.BlockSpec((RB, D), lambda i: (i, 0)),
        out_shape=jax.ShapeDtypeStruct((N, D), jnp.float32),
    )


def kernel(lane_features, suc_0, n_hop_pre_0, pre_0, n_hop_suc_0,
           suc_1, n_hop_pre_1, pre_1, n_hop_suc_1,
           W_center, W_pre, W_suc, gn1_gamma, gn1_beta,
           W_center2, gn2_gamma, gn2_beta):
    N, D = lane_features.shape
    E = suc_0.shape[0]
    NB = W_center.shape[0]

    srcs = jnp.stack([n_hop_pre_0, n_hop_suc_0, n_hop_pre_1, n_hop_suc_1])
    dsts = jnp.stack([suc_0, pre_0, suc_1, pre_1])
    srcs = srcs.astype(jnp.int32).reshape(NUM_LISTS, NUM_SUBCORES, -1, CHUNK)
    dsts = dsts.astype(jnp.int32).reshape(NUM_LISTS, NUM_SUBCORES, -1, CHUNK)
    zeros_h = jnp.zeros((128, QW), jnp.float32)

    sc = _build_sc(N, D, E)
    tc = _build_tc(N, D)

    x = lane_features
    for b in range(NB):
        wcat = jnp.concatenate([W_center[b].T, W_pre[0, b].T, W_suc[0, b].T,
                                W_pre[1, b].T, W_suc[1, b].T], axis=0)
        xq = jnp.swapaxes(x.reshape(N, NQ, QW), 0, 1)
        accs = sc(xq, srcs, dsts, zeros_h)
        x = tc(x, accs, wcat, W_center2[b].T,
               gn1_gamma[b][None], gn1_beta[b][None],
               gn2_gamma[b][None], gn2_beta[b][None])
    return x
